# baseline (speedup 1.0000x reference)
import functools

import jax
import jax.numpy as jnp
from jax import lax
from jax.experimental import pallas as pl
from jax.experimental.pallas import tpu as pltpu
from jax.experimental.pallas import tpu_sc as plsc

NC = 2
NS = 16
NW = NC * NS
CHUNK = 128
L = 16


def _sc_gather_fn(B, title_dim):
    rows_per_w = B // NW
    n_chunks = rows_per_w // CHUNK
    mesh = plsc.VectorSubcoreMesh(core_axis_name="c", subcore_axis_name="s")

    @functools.partial(
        pl.kernel,
        mesh=mesh,
        out_type=(
            jax.ShapeDtypeStruct((B, title_dim), jnp.float32),
            jax.ShapeDtypeStruct((B, 128), jnp.float32),
        ),
        scratch_types=[
            pltpu.VMEM((rows_per_w,), jnp.int32),
            pltpu.VMEM((rows_per_w,), jnp.int32),
            pltpu.VMEM((rows_per_w,), jnp.int32),
            pltpu.VMEM((L,), jnp.float32),
            pltpu.VMEM((CHUNK, title_dim), jnp.float32),
            pltpu.VMEM((CHUNK, title_dim), jnp.float32),
            pltpu.VMEM((CHUNK, 128), jnp.float32),
            pltpu.SemaphoreType.DMA,
            pltpu.SemaphoreType.DMA,
        ],
    )
    def sc_gather(news_hbm, topic_i_hbm, subtopic_i_hbm,
                  title_hbm, abstract_hbm, tsa_hbm, tsb_hbm, aw_hbm,
                  emb_hbm, comb_hbm,
                  idx_n, idx_t, idx_s, aws, tb, ab, cb, sem, sem_t):
        wid = lax.axis_index("s") * NC + lax.axis_index("c")
        base0 = wid * rows_per_w
        pltpu.sync_copy(news_hbm.at[pl.ds(base0, rows_per_w)], idx_n)
        pltpu.sync_copy(topic_i_hbm.at[pl.ds(base0, rows_per_w)], idx_t)
        pltpu.sync_copy(subtopic_i_hbm.at[pl.ds(base0, rows_per_w)], idx_s)
        pltpu.sync_copy(aw_hbm, aws)
        ev = jnp.exp(aws[...])
        e0 = ev[0]
        e1 = ev[1]
        zf = jnp.zeros((L,), jnp.float32)
        rv = (zf + 1.0) / (zf + (e0 + e1))
        a0 = (zf + e0) * rv
        a1 = (zf + e1) * rv

        def chunk_body(c, carry):
            lo = pl.multiple_of(c * CHUNK, CHUNK)
            base = base0 + lo
            g1 = pltpu.async_copy(title_hbm.at[idx_n.at[pl.ds(lo, CHUNK)]],
                                  tb, sem)
            g2 = pltpu.async_copy(abstract_hbm.at[idx_n.at[pl.ds(lo, CHUNK)]],
                                  ab, sem)
            g3 = pltpu.async_copy(tsa_hbm.at[idx_t.at[pl.ds(lo, CHUNK)]],
                                  cb, sem_t)
            g3.wait()
            g4 = pltpu.async_copy(tsb_hbm.at[idx_s.at[pl.ds(lo, CHUNK)]],
                                  cb, sem_t, add=True)
            g1.wait()
            g2.wait()

            def row_body(r, rc):
                for j in range(title_dim // L):
                    sl = pl.ds(j * L, L)
                    tb[r, sl] = a0 * tb[r, sl] + a1 * ab[r, sl]
                return rc

            lax.fori_loop(0, CHUNK, row_body, 0)
            g4.wait()
            s1 = pltpu.async_copy(tb, emb_hbm.at[pl.ds(base, CHUNK)], sem)
            s2 = pltpu.async_copy(cb, comb_hbm.at[pl.ds(base, CHUNK)], sem)
            s1.wait()
            s2.wait()
            return carry

        lax.fori_loop(0, n_chunks, chunk_body, 0)

    return sc_gather


def _tc_body(w_ref, b_ref, emb_ref, comb_ref, out_ref):
    w = w_ref[...]
    dn = (((1,), (1,)), ((), ()))
    acc = lax.dot_general(emb_ref[...], w[:, 0:128], dn,
                          preferred_element_type=jnp.float32)
    acc = acc + lax.dot_general(comb_ref[...], w[:, 128:256], dn,
                                preferred_element_type=jnp.float32)
    out_ref[...] = acc + b_ref[...]


def _tc_matmul(W, b2d, emb, comb):
    B = emb.shape[0]
    RB = 2048
    grid = (B // RB,)
    return pl.pallas_call(
        _tc_body,
        grid=grid,
        in_specs=[
            pl.BlockSpec((256, 256), lambda i: (0, 0)),
            pl.BlockSpec((1, 256), lambda i: (0, 0)),
            pl.BlockSpec((RB, 128), lambda i: (i, 0)),
            pl.BlockSpec((RB, 128), lambda i: (i, 0)),
        ],
        out_specs=pl.BlockSpec((RB, 256), lambda i: (i, 0)),
        out_shape=jax.ShapeDtypeStruct((B, 256), jnp.float32),
    )(W, b2d, emb, comb)


def kernel(news, news_topic, news_subtopic, title_vectors, abstract_vectors,
           topic_embed, subtopic_embed, attention_weight, W, b):
    news = news.astype(jnp.int32)
    news_topic = news_topic.astype(jnp.int32)
    news_subtopic = news_subtopic.astype(jnp.int32)
    B = news.shape[0]
    title_dim = title_vectors.shape[1]
    topic_dim = topic_embed.shape[1]

    tsa = jnp.pad(topic_embed, ((0, 0), (0, 128 - topic_dim)))
    tsb = jnp.pad(subtopic_embed, ((0, 0), (128 - topic_dim, 0)))
    aw16 = jnp.pad(attention_weight.astype(jnp.float32), (0, L - 2))

    sc = _sc_gather_fn(B, title_dim)
    emb, comb = sc(news, news_topic, news_subtopic,
                   title_vectors, abstract_vectors, tsa, tsb, aw16)

    return _tc_matmul(W, b.reshape(1, -1), emb, comb)


# speedup vs baseline: 1.0442x; 1.0442x over previous
import functools

import jax
import jax.numpy as jnp
from jax import lax
from jax.experimental import pallas as pl
from jax.experimental.pallas import tpu as pltpu
from jax.experimental.pallas import tpu_sc as plsc

NC = 2
NS = 16
NW = NC * NS
CHUNK = 128
L = 16


def _sc_gather_fn(B, title_dim):
    rows_per_w = B // NW
    n_chunks = rows_per_w // CHUNK
    mesh = plsc.VectorSubcoreMesh(core_axis_name="c", subcore_axis_name="s")

    @functools.partial(
        pl.kernel,
        mesh=mesh,
        out_type=(
            jax.ShapeDtypeStruct((B, title_dim), jnp.float32),
            jax.ShapeDtypeStruct((B, 128), jnp.float32),
        ),
        scratch_types=[
            pltpu.VMEM((rows_per_w,), jnp.int32),
            pltpu.VMEM((rows_per_w,), jnp.int32),
            pltpu.VMEM((rows_per_w,), jnp.int32),
            pltpu.VMEM((L,), jnp.float32),
            pltpu.VMEM((CHUNK, title_dim), jnp.float32),
            pltpu.VMEM((CHUNK, title_dim), jnp.float32),
            pltpu.VMEM((CHUNK, 128), jnp.float32),
            pltpu.SemaphoreType.DMA,
            pltpu.SemaphoreType.DMA,
        ],
    )
    def sc_gather(news_hbm, topic_i_hbm, subtopic_i_hbm,
                  title_hbm, abstract_hbm, tsa_hbm, tsb_hbm, aw_hbm,
                  emb_hbm, comb_hbm,
                  idx_n, idx_t, idx_s, aws, tb, ab, cb, sem, sem_t):
        wid = lax.axis_index("s") * NC + lax.axis_index("c")
        base0 = wid * rows_per_w
        pltpu.sync_copy(news_hbm.at[pl.ds(base0, rows_per_w)], idx_n)
        pltpu.sync_copy(topic_i_hbm.at[pl.ds(base0, rows_per_w)], idx_t)
        pltpu.sync_copy(subtopic_i_hbm.at[pl.ds(base0, rows_per_w)], idx_s)
        pltpu.sync_copy(aw_hbm, aws)
        ev = jnp.exp(aws[...])
        e0 = ev[0]
        e1 = ev[1]
        zf = jnp.zeros((L,), jnp.float32)
        rv = (zf + 1.0) / (zf + (e0 + e1))
        a0 = (zf + e0) * rv
        a1 = (zf + e1) * rv

        def chunk_body(c, carry):
            lo = pl.multiple_of(c * CHUNK, CHUNK)
            base = base0 + lo
            g1 = pltpu.async_copy(title_hbm.at[idx_n.at[pl.ds(lo, CHUNK)]],
                                  tb, sem)
            g2 = pltpu.async_copy(abstract_hbm.at[idx_n.at[pl.ds(lo, CHUNK)]],
                                  ab, sem)
            g3 = pltpu.async_copy(tsa_hbm.at[idx_t.at[pl.ds(lo, CHUNK)]],
                                  cb, sem_t)
            g3.wait()
            g4 = pltpu.async_copy(tsb_hbm.at[idx_s.at[pl.ds(lo, CHUNK)]],
                                  cb, sem_t, add=True)
            g1.wait()
            g2.wait()

            def row_body(r, rc):
                for j in range(title_dim // L):
                    sl = pl.ds(j * L, L)
                    tb[r, sl] = a0 * tb[r, sl] + a1 * ab[r, sl]
                return rc

            lax.fori_loop(0, CHUNK, row_body, 0)
            g4.wait()
            s1 = pltpu.async_copy(tb, emb_hbm.at[pl.ds(base, CHUNK)], sem)
            s2 = pltpu.async_copy(cb, comb_hbm.at[pl.ds(base, CHUNK)], sem)
            s1.wait()
            s2.wait()
            return carry

        lax.fori_loop(0, n_chunks, chunk_body, 0)

    return sc_gather


def _tc_body(w_ref, b_ref, emb_ref, comb_ref, out_ref):
    w = w_ref[...]
    dn = (((1,), (1,)), ((), ()))
    acc = lax.dot_general(emb_ref[...], w[:, 0:128], dn,
                          preferred_element_type=jnp.float32)
    acc = acc + lax.dot_general(comb_ref[...], w[:, 128:256], dn,
                                preferred_element_type=jnp.float32)
    out_ref[...] = acc + b_ref[...]


def _tc_matmul(W, b2d, emb, comb):
    B = emb.shape[0]
    RB = 8192
    grid = (B // RB,)
    return pl.pallas_call(
        _tc_body,
        grid=grid,
        in_specs=[
            pl.BlockSpec((256, 256), lambda i: (0, 0)),
            pl.BlockSpec((1, 256), lambda i: (0, 0)),
            pl.BlockSpec((RB, 128), lambda i: (i, 0)),
            pl.BlockSpec((RB, 128), lambda i: (i, 0)),
        ],
        out_specs=pl.BlockSpec((RB, 256), lambda i: (i, 0)),
        out_shape=jax.ShapeDtypeStruct((B, 256), jnp.float32),
    )(W, b2d, emb, comb)


def kernel(news, news_topic, news_subtopic, title_vectors, abstract_vectors,
           topic_embed, subtopic_embed, attention_weight, W, b):
    news = news.astype(jnp.int32)
    news_topic = news_topic.astype(jnp.int32)
    news_subtopic = news_subtopic.astype(jnp.int32)
    B = news.shape[0]
    title_dim = title_vectors.shape[1]
    topic_dim = topic_embed.shape[1]

    tsa = jnp.pad(topic_embed, ((0, 0), (0, 128 - topic_dim)))
    tsb = jnp.pad(subtopic_embed, ((0, 0), (128 - topic_dim, 0)))
    aw16 = jnp.pad(attention_weight.astype(jnp.float32), (0, L - 2))

    sc = _sc_gather_fn(B, title_dim)
    emb, comb = sc(news, news_topic, news_subtopic,
                   title_vectors, abstract_vectors, tsa, tsb, aw16)

    return _tc_matmul(W, b.reshape(1, -1), emb, comb)
